# baseline (speedup 1.0000x reference)
import functools

import jax
import jax.numpy as jnp
from jax.experimental import pallas as pl
from jax.experimental.pallas import tpu as pltpu
from jax.experimental.pallas import tpu_sc as plsc

_NUM_TOKENS = 20
_EMBEDDING_DIM = 64
_ROW = _NUM_TOKENS * _EMBEDDING_DIM


def kernel(memory, forgetting_factor, task_id):
    del forgetting_factor
    tid = jnp.asarray(task_id, jnp.int32)
    mem3 = memory.reshape(memory.shape[0], _NUM_TOKENS, _EMBEDDING_DIM)
    mesh = plsc.ScalarSubcoreMesh(axis_name="a", num_cores=1)

    @functools.partial(
        pl.kernel,
        out_type=jax.ShapeDtypeStruct((_NUM_TOKENS, _EMBEDDING_DIM), jnp.float32),
        mesh=mesh,
        scratch_types=[pltpu.SemaphoreType.DMA],
    )
    def run(mem_hbm, out_hbm, sem):
        pltpu.async_copy(mem_hbm.at[tid], out_hbm, sem).wait()

    return run(mem3)


# speedup vs baseline: 58.6211x; 58.6211x over previous
import functools

import jax
import jax.numpy as jnp
from jax.experimental import pallas as pl
from jax.experimental.pallas import tpu as pltpu
from jax.experimental.pallas import tpu_sc as plsc

_NUM_TOKENS = 20
_EMBEDDING_DIM = 64
_ROW = _NUM_TOKENS * _EMBEDDING_DIM


def kernel(memory, forgetting_factor, task_id):
    del forgetting_factor
    tid = jnp.asarray(task_id, jnp.int32)
    mesh = plsc.ScalarSubcoreMesh(axis_name="a", num_cores=2)

    @functools.partial(
        pl.kernel,
        out_type=jax.ShapeDtypeStruct((1, _ROW), jnp.float32),
        mesh=mesh,
        scratch_types=[pltpu.SemaphoreType.DMA],
    )
    def run(mem_hbm, out_hbm, sem):
        pltpu.async_copy(mem_hbm.at[pl.ds(tid, 1)], out_hbm, sem).wait()

    out = run(memory)
    return out.reshape(_NUM_TOKENS, _EMBEDDING_DIM)


# speedup vs baseline: 64.0811x; 1.0931x over previous
import functools

import jax
import jax.numpy as jnp
from jax.experimental import pallas as pl
from jax.experimental.pallas import tpu as pltpu
from jax.experimental.pallas import tpu_sc as plsc

_NUM_TOKENS = 20
_EMBEDDING_DIM = 64
_ROW = _NUM_TOKENS * _EMBEDDING_DIM


def kernel(memory, forgetting_factor, task_id):
    del forgetting_factor
    tid = jnp.asarray(task_id, jnp.int32)
    mesh = plsc.ScalarSubcoreMesh(axis_name="a", num_cores=1)

    @functools.partial(
        pl.kernel,
        out_type=jax.ShapeDtypeStruct((1, _ROW), jnp.float32),
        mesh=mesh,
        scratch_types=[pltpu.SemaphoreType.DMA],
        compiler_params=pltpu.CompilerParams(skip_device_barrier=True),
    )
    def run(mem_hbm, out_hbm, sem):
        pltpu.async_copy(mem_hbm.at[pl.ds(tid, 1)], out_hbm, sem).wait()

    out = run(memory)
    return out.reshape(_NUM_TOKENS, _EMBEDDING_DIM)
